# baseline (speedup 1.0000x reference)
import functools

import jax
import jax.numpy as jnp
import numpy as np
from jax import lax
from jax.experimental import pallas as pl
from jax.experimental.pallas import tpu as pltpu
from jax.experimental.pallas import tpu_sc as plsc

GRID_LEVELS = 16
MAX_GRID = 2 ** 16
FEAT_DIM = 16
COARSE = 16
FINE = 512
_B32 = np.float32(2.0) ** (np.float32(np.log2(FINE / float(COARSE))) / np.float32(GRID_LEVELS - 1))
_NL = [float(np.floor(np.float32(COARSE) * _B32 ** np.float32(l))) for l in range(GRID_LEVELS)]
_FY = 2654435761 % 65536
_FZ = 805459861 % 65536

NC, NS, L = 2, 16, 16
NW = NC * NS
N_POINTS = 512 * 8 * 8
P_PER_W = N_POINTS // NW
CH = 256
BLK = 16
N_CHUNKS = P_PER_W // CH
ITERS = (CH // BLK) * GRID_LEVELS


SLOTS = 4
AHEAD = 3


def _sc_hashgrid_body(coords_hbm, table_hbm, nl_hbm, feat_hbm,
                      coords_v, nl_v, idx_vs, rows_vs, feat_v, sems):
    wid = lax.axis_index("s") * NC + lax.axis_index("c")
    wbase = wid * P_PER_W
    pltpu.sync_copy(nl_hbm, nl_v)
    iota = lax.iota(jnp.int32, L)

    def coords_frac(i):
        if isinstance(i, int):
            ob = (i >> 4) * BLK
            l = i & 15
            nl = jnp.full((L,), np.float32(_NL[l]), jnp.float32)
        else:
            b = i >> jnp.int32(4)
            l = i & jnp.int32(15)
            ob = b * jnp.int32(BLK)
            nl = plsc.load_gather(nl_v, [jnp.full((L,), l, jnp.int32)])
        px = coords_v[0, pl.ds(ob, L)]
        py = coords_v[1, pl.ds(ob, L)]
        pz = coords_v[2, pl.ds(ob, L)]
        xsx = px * nl
        xsy = py * nl
        xsz = pz * nl
        lx = xsx.astype(jnp.int32)
        ly = xsy.astype(jnp.int32)
        lz = xsz.astype(jnp.int32)
        fx = xsx - lx.astype(jnp.float32)
        fy = xsy - ly.astype(jnp.float32)
        fz = xsz - lz.astype(jnp.float32)
        return ob, l, lx, ly, lz, fx, fy, fz

    def compute_fire(i, idx_v, rows_v, sem):
        _, _, lx, ly, lz, _, _, _ = coords_frac(i)
        hy0 = ly * _FY
        hz0 = lz * _FZ
        hy1 = hy0 + _FY
        hz1 = hz0 + _FZ
        hx1 = lx + 1
        mask = jnp.int32(0xFFFF)
        idx_v[pl.ds(0 * L, L)] = (lx ^ hy0 ^ hz0) & mask
        idx_v[pl.ds(1 * L, L)] = (lx ^ hy0 ^ hz1) & mask
        idx_v[pl.ds(2 * L, L)] = (lx ^ hy1 ^ hz0) & mask
        idx_v[pl.ds(3 * L, L)] = (lx ^ hy1 ^ hz1) & mask
        idx_v[pl.ds(4 * L, L)] = (hx1 ^ hy0 ^ hz0) & mask
        idx_v[pl.ds(5 * L, L)] = (hx1 ^ hy0 ^ hz1) & mask
        idx_v[pl.ds(6 * L, L)] = (hx1 ^ hy1 ^ hz0) & mask
        idx_v[pl.ds(7 * L, L)] = (hx1 ^ hy1 ^ hz1) & mask
        pltpu.async_copy(table_hbm.at[idx_v], rows_v, sem)

    def wait_gather(idx_v, rows_v, sem):
        pltpu.make_async_copy(table_hbm.at[idx_v], rows_v, sem).wait()

    def interp(i, rows_v):
        ob, l, _, _, _, fx, fy, fz = coords_frac(i)
        gx = jnp.float32(1.0) - fx
        gy = jnp.float32(1.0) - fy
        gz = jnp.float32(1.0) - fz
        w = [gx * gy * gz, gx * gy * fz, gx * fy * gz, gx * fy * fz,
             fx * gy * gz, fx * gy * fz, fx * fy * gz, fx * fy * fz]
        out_base = (iota + ob) * 256 + l * FEAT_DIM
        for f in range(FEAT_DIM):
            col = jnp.full((L,), f, jnp.int32)
            acc = w[0] * plsc.load_gather(rows_v, [iota, col])
            for c in range(1, 8):
                acc = acc + w[c] * plsc.load_gather(rows_v, [iota + c * L, col])
            plsc.store_scatter(feat_v, [out_base + f], acc)

    def group_body(j, carry):
        for b in range(SLOTS):
            i = j * jnp.int32(SLOTS) + jnp.int32(b)
            sf = (b + AHEAD) % SLOTS
            wait_gather(idx_vs[b], rows_vs[b], sems[b])
            compute_fire(i + jnp.int32(AHEAD), idx_vs[sf], rows_vs[sf],
                         sems[sf])
            interp(i, rows_vs[b])
        return carry

    n_groups = ITERS // SLOTS

    def chunk_body(k, carry):
        base = wbase + k * jnp.int32(CH)
        pltpu.sync_copy(coords_hbm.at[:, pl.ds(base, CH)], coords_v)
        for p in range(AHEAD):
            compute_fire(p, idx_vs[p], rows_vs[p], sems[p])
        lax.fori_loop(jnp.int32(0), jnp.int32(n_groups - 1), group_body,
                      jnp.int32(0), unroll=False)
        for b in range(SLOTS):
            i = (n_groups - 1) * SLOTS + b
            sf = (b + AHEAD) % SLOTS
            wait_gather(idx_vs[b], rows_vs[b], sems[b])
            if i + AHEAD < ITERS:
                compute_fire(i + AHEAD, idx_vs[sf], rows_vs[sf], sems[sf])
            interp(i, rows_vs[b])
        pltpu.sync_copy(feat_v,
                        feat_hbm.at[pl.ds(base * jnp.int32(256), CH * 256)])
        return carry

    lax.fori_loop(jnp.int32(0), jnp.int32(N_CHUNKS), chunk_body,
                  jnp.int32(0), unroll=False)


def _sc_hashgrid(coords, table, nl_arr):
    mesh = plsc.VectorSubcoreMesh(core_axis_name="c", subcore_axis_name="s",
                                  num_cores=NC, num_subcores=NS)
    f = pl.kernel(
        _sc_hashgrid_body,
        out_type=jax.ShapeDtypeStruct((N_POINTS * 256,), jnp.float32),
        mesh=mesh,
        scratch_types=[
            pltpu.VMEM((3, CH), jnp.float32),
            pltpu.VMEM((GRID_LEVELS,), jnp.float32),
            [pltpu.VMEM((8 * L,), jnp.int32) for _ in range(SLOTS)],
            [pltpu.VMEM((8 * L, FEAT_DIM), jnp.float32) for _ in range(SLOTS)],
            pltpu.VMEM((CH * 256,), jnp.float32),
            [pltpu.SemaphoreType.DMA for _ in range(SLOTS)],
        ],
        compiler_params=pltpu.CompilerParams(needs_layout_passes=False,
                                             use_tc_tiling_on_sc=False),
    )
    return f(coords, table, nl_arr)


def _posenc_body(x_ref, out_ref):
    v = x_ref[...]
    pi = jnp.float32(np.pi)
    s = jnp.sin(v * pi)
    c = jnp.cos(v * pi)
    out_ref[0] = v
    for i in range(6):
        out_ref[1 + 2 * i] = s
        out_ref[2 + 2 * i] = c
        if i < 5:
            s, c = jnp.float32(2.0) * s * c, jnp.float32(1.0) - jnp.float32(2.0) * s * s


def _posenc(x_flat):
    rows = x_flat.shape[0] // 128
    return pl.pallas_call(
        _posenc_body,
        out_shape=jax.ShapeDtypeStruct((13, rows, 128), jnp.float32),
    )(x_flat.reshape(rows, 128))


def kernel(x, t, mask, table_bank):
    msk = jnp.squeeze(mask)
    num_keep = msk.shape[0] - 1
    keep_idx = jnp.argsort(msk == 0)[:num_keep]
    x_sel = jnp.take(x, keep_idx, axis=-1)
    tidx = jnp.argmax(msk == 0)
    table = jnp.take(table_bank, tidx, axis=0).astype(jnp.float32)
    N, H, W = x_sel.shape[0], x_sel.shape[1], x_sel.shape[2]
    tt = jnp.broadcast_to(t[:, None, None, :], (N, H, W, 1)).astype(jnp.float32)
    x_t = jnp.concatenate([x_sel.astype(jnp.float32), tt], axis=-1)
    xt2 = x_t.reshape(N_POINTS, 3)
    coords = xt2.T.copy()
    nl_arr = jnp.asarray(_NL, dtype=jnp.float32)

    feat = _sc_hashgrid(coords, table, nl_arr).reshape(N_POINTS, 256)
    enc = _posenc(xt2.reshape(-1))
    enc = enc.reshape(13, N_POINTS, 3).transpose(1, 0, 2).reshape(N_POINTS, 39)
    latent = jnp.concatenate([feat, enc], axis=-1)
    return latent.reshape(N, H, W, 256 + 39)


# speedup vs baseline: 1.1464x; 1.1464x over previous
import functools

import jax
import jax.numpy as jnp
import numpy as np
from jax import lax
from jax.experimental import pallas as pl
from jax.experimental.pallas import tpu as pltpu
from jax.experimental.pallas import tpu_sc as plsc

GRID_LEVELS = 16
MAX_GRID = 2 ** 16
FEAT_DIM = 16
COARSE = 16
FINE = 512
_B32 = np.float32(2.0) ** (np.float32(np.log2(FINE / float(COARSE))) / np.float32(GRID_LEVELS - 1))
_NL = [float(np.floor(np.float32(COARSE) * _B32 ** np.float32(l))) for l in range(GRID_LEVELS)]
_FY = 2654435761 % 65536
_FZ = 805459861 % 65536

NC, NS, L = 2, 16, 16
NW = NC * NS
N_POINTS = 512 * 8 * 8
P_PER_W = N_POINTS // NW
CH = 256
BLK = 16
N_CHUNKS = P_PER_W // CH
ITERS = (CH // BLK) * GRID_LEVELS


SLOTS = 2
AHEAD = 1


def _sc_hashgrid_body(coords_hbm, table_hbm, nl_hbm, feat_hbm,
                      coords_v, nl_v, idx_vs, pidx_vs, rows_vs, feat_v, sems):
    wid = lax.axis_index("s") * NC + lax.axis_index("c")
    wbase = wid * P_PER_W
    pltpu.sync_copy(nl_hbm, nl_v)
    iota = lax.iota(jnp.int32, L)

    def coords_frac(i):
        if isinstance(i, int):
            ob = (i >> 4) * BLK
            l = i & 15
            nl = jnp.full((L,), np.float32(_NL[l]), jnp.float32)
        else:
            b = i >> jnp.int32(4)
            l = i & jnp.int32(15)
            ob = b * jnp.int32(BLK)
            nl = plsc.load_gather(nl_v, [jnp.full((L,), l, jnp.int32)])
        px = coords_v[0, pl.ds(ob, L)]
        py = coords_v[1, pl.ds(ob, L)]
        pz = coords_v[2, pl.ds(ob, L)]
        xsx = px * nl
        xsy = py * nl
        xsz = pz * nl
        lx = xsx.astype(jnp.int32)
        ly = xsy.astype(jnp.int32)
        lz = xsz.astype(jnp.int32)
        fx = xsx - lx.astype(jnp.float32)
        fy = xsy - ly.astype(jnp.float32)
        fz = xsz - lz.astype(jnp.float32)
        return ob, l, lx, ly, lz, fx, fy, fz

    def compute_fire(i, idx_v, pidx_v, rows_v, sem):
        _, _, lx, ly, lz, _, _, _ = coords_frac(i)
        hy0 = ly * _FY
        hz0 = lz * _FZ
        hy1 = hy0 + _FY
        hz1 = hz0 + _FZ
        hx1 = lx + 1
        mask = jnp.int32(0xFFFF)
        idx_v[pl.ds(0 * L, L)] = (lx ^ hy0 ^ hz0) & mask
        idx_v[pl.ds(1 * L, L)] = (lx ^ hy0 ^ hz1) & mask
        idx_v[pl.ds(2 * L, L)] = (lx ^ hy1 ^ hz0) & mask
        idx_v[pl.ds(3 * L, L)] = (lx ^ hy1 ^ hz1) & mask
        idx_v[pl.ds(4 * L, L)] = (hx1 ^ hy0 ^ hz0) & mask
        idx_v[pl.ds(5 * L, L)] = (hx1 ^ hy0 ^ hz1) & mask
        idx_v[pl.ds(6 * L, L)] = (hx1 ^ hy1 ^ hz0) & mask
        idx_v[pl.ds(7 * L, L)] = (hx1 ^ hy1 ^ hz1) & mask
        one = jnp.int32(1)
        for c in range(8):
            pidx_v[pl.ds(c * L, L)] = idx_v[pl.ds(c * L, L)] >> one
        pltpu.async_copy(table_hbm.at[pidx_v], rows_v, sem)

    def wait_gather(pidx_v, rows_v, sem):
        pltpu.make_async_copy(table_hbm.at[pidx_v], rows_v, sem).wait()

    def interp(i, idx_v, rows_v):
        ob, l, _, _, _, fx, fy, fz = coords_frac(i)
        gx = jnp.float32(1.0) - fx
        gy = jnp.float32(1.0) - fy
        gz = jnp.float32(1.0) - fz
        w = [gx * gy * gz, gx * gy * fz, gx * fy * gz, gx * fy * fz,
             fx * gy * gz, fx * gy * fz, fx * fy * gz, fx * fy * fz]
        out_base = (iota + ob) * 256 + l * FEAT_DIM
        one = jnp.int32(1)
        colb = [(idx_v[pl.ds(c * L, L)] & one) * jnp.int32(FEAT_DIM)
                for c in range(8)]
        for f in range(FEAT_DIM):
            acc = w[0] * plsc.load_gather(rows_v, [iota, colb[0] + f])
            for c in range(1, 8):
                acc = acc + w[c] * plsc.load_gather(
                    rows_v, [iota + c * L, colb[c] + f])
            plsc.store_scatter(feat_v, [out_base + f], acc)

    def group_body(j, carry):
        for b in range(SLOTS):
            i = j * jnp.int32(SLOTS) + jnp.int32(b)
            sf = (b + AHEAD) % SLOTS
            wait_gather(pidx_vs[b], rows_vs[b], sems[b])
            compute_fire(i + jnp.int32(AHEAD), idx_vs[sf], pidx_vs[sf],
                         rows_vs[sf], sems[sf])
            interp(i, idx_vs[b], rows_vs[b])
        return carry

    n_groups = ITERS // SLOTS

    def chunk_body(k, carry):
        base = wbase + k * jnp.int32(CH)
        pltpu.sync_copy(coords_hbm.at[:, pl.ds(base, CH)], coords_v)
        for p in range(AHEAD):
            compute_fire(p, idx_vs[p], pidx_vs[p], rows_vs[p], sems[p])
        lax.fori_loop(jnp.int32(0), jnp.int32(n_groups - 1), group_body,
                      jnp.int32(0), unroll=False)
        for b in range(SLOTS):
            i = (n_groups - 1) * SLOTS + b
            sf = (b + AHEAD) % SLOTS
            wait_gather(pidx_vs[b], rows_vs[b], sems[b])
            if i + AHEAD < ITERS:
                compute_fire(i + AHEAD, idx_vs[sf], pidx_vs[sf], rows_vs[sf],
                             sems[sf])
            interp(i, idx_vs[b], rows_vs[b])
        pltpu.sync_copy(feat_v,
                        feat_hbm.at[pl.ds(base * jnp.int32(256), CH * 256)])
        return carry

    lax.fori_loop(jnp.int32(0), jnp.int32(N_CHUNKS), chunk_body,
                  jnp.int32(0), unroll=False)


def _sc_hashgrid(coords, table, nl_arr):
    mesh = plsc.VectorSubcoreMesh(core_axis_name="c", subcore_axis_name="s",
                                  num_cores=NC, num_subcores=NS)
    f = pl.kernel(
        _sc_hashgrid_body,
        out_type=jax.ShapeDtypeStruct((N_POINTS * 256,), jnp.float32),
        mesh=mesh,
        scratch_types=[
            pltpu.VMEM((3, CH), jnp.float32),
            pltpu.VMEM((GRID_LEVELS,), jnp.float32),
            [pltpu.VMEM((8 * L,), jnp.int32) for _ in range(SLOTS)],
            [pltpu.VMEM((8 * L,), jnp.int32) for _ in range(SLOTS)],
            [pltpu.VMEM((8 * L, 2 * FEAT_DIM), jnp.float32)
             for _ in range(SLOTS)],
            pltpu.VMEM((CH * 256,), jnp.float32),
            [pltpu.SemaphoreType.DMA for _ in range(SLOTS)],
        ],
        compiler_params=pltpu.CompilerParams(needs_layout_passes=False,
                                             use_tc_tiling_on_sc=False),
    )
    return f(coords, table, nl_arr)


def _posenc_body(x_ref, out_ref):
    v = x_ref[...]
    pi = jnp.float32(np.pi)
    s = jnp.sin(v * pi)
    c = jnp.cos(v * pi)
    out_ref[0] = v
    for i in range(6):
        out_ref[1 + 2 * i] = s
        out_ref[2 + 2 * i] = c
        if i < 5:
            s, c = jnp.float32(2.0) * s * c, jnp.float32(1.0) - jnp.float32(2.0) * s * s


def _posenc(x_flat):
    rows = x_flat.shape[0] // 128
    return pl.pallas_call(
        _posenc_body,
        out_shape=jax.ShapeDtypeStruct((13, rows, 128), jnp.float32),
    )(x_flat.reshape(rows, 128))


def kernel(x, t, mask, table_bank):
    msk = jnp.squeeze(mask)
    num_keep = msk.shape[0] - 1
    keep_idx = jnp.argsort(msk == 0)[:num_keep]
    x_sel = jnp.take(x, keep_idx, axis=-1)
    tidx = jnp.argmax(msk == 0)
    table = jnp.take(table_bank, tidx, axis=0).astype(jnp.float32)
    N, H, W = x_sel.shape[0], x_sel.shape[1], x_sel.shape[2]
    tt = jnp.broadcast_to(t[:, None, None, :], (N, H, W, 1)).astype(jnp.float32)
    x_t = jnp.concatenate([x_sel.astype(jnp.float32), tt], axis=-1)
    xt2 = x_t.reshape(N_POINTS, 3)
    coords = xt2.T.copy()
    nl_arr = jnp.asarray(_NL, dtype=jnp.float32)

    table2 = table.reshape(MAX_GRID // 2, 2 * FEAT_DIM)
    feat = _sc_hashgrid(coords, table2, nl_arr).reshape(N_POINTS, 256)
    enc = _posenc(xt2.reshape(-1))
    enc = enc.reshape(13, N_POINTS, 3).transpose(1, 0, 2).reshape(N_POINTS, 39)
    latent = jnp.concatenate([feat, enc], axis=-1)
    return latent.reshape(N, H, W, 256 + 39)


# speedup vs baseline: 1.1665x; 1.0175x over previous
import functools

import jax
import jax.numpy as jnp
import numpy as np
from jax import lax
from jax.experimental import pallas as pl
from jax.experimental.pallas import tpu as pltpu
from jax.experimental.pallas import tpu_sc as plsc

GRID_LEVELS = 16
MAX_GRID = 2 ** 16
FEAT_DIM = 16
COARSE = 16
FINE = 512
_B32 = np.float32(2.0) ** (np.float32(np.log2(FINE / float(COARSE))) / np.float32(GRID_LEVELS - 1))
_NL = [float(np.floor(np.float32(COARSE) * _B32 ** np.float32(l))) for l in range(GRID_LEVELS)]
_FY = 2654435761 % 65536
_FZ = 805459861 % 65536

NC, NS, L = 2, 16, 16
NW = NC * NS
N_POINTS = 512 * 8 * 8
P_PER_W = N_POINTS // NW
CH = 256
BLK = 16
N_CHUNKS = P_PER_W // CH
ITERS = (CH // BLK) * GRID_LEVELS


SLOTS = 2
AHEAD = 1


def _sc_hashgrid_body(coords_hbm, table_hbm, nl_hbm, feat_hbm,
                      coords_v, nl_v, idx_vs, pidx_vs, rows_vs, feat_v, sems):
    wid = lax.axis_index("s") * NC + lax.axis_index("c")
    wbase = wid * P_PER_W
    pltpu.sync_copy(nl_hbm, nl_v)
    iota = lax.iota(jnp.int32, L)

    def coords_frac(i):
        if isinstance(i, int):
            ob = (i >> 4) * BLK
            l = (jnp.int32(i) + wid) & jnp.int32(15)
        else:
            ob = (i >> jnp.int32(4)) * jnp.int32(BLK)
            l = (i + wid) & jnp.int32(15)
        nl = plsc.load_gather(nl_v, [jnp.full((L,), l, jnp.int32)])
        px = coords_v[0, pl.ds(ob, L)]
        py = coords_v[1, pl.ds(ob, L)]
        pz = coords_v[2, pl.ds(ob, L)]
        xsx = px * nl
        xsy = py * nl
        xsz = pz * nl
        lx = xsx.astype(jnp.int32)
        ly = xsy.astype(jnp.int32)
        lz = xsz.astype(jnp.int32)
        fx = xsx - lx.astype(jnp.float32)
        fy = xsy - ly.astype(jnp.float32)
        fz = xsz - lz.astype(jnp.float32)
        return ob, l, lx, ly, lz, fx, fy, fz

    def compute_fire(i, idx_v, pidx_v, rows_v, sem):
        _, _, lx, ly, lz, _, _, _ = coords_frac(i)
        hy0 = ly * _FY
        hz0 = lz * _FZ
        hy1 = hy0 + _FY
        hz1 = hz0 + _FZ
        hx1 = lx + 1
        mask = jnp.int32(0xFFFF)
        idx_v[pl.ds(0 * L, L)] = (lx ^ hy0 ^ hz0) & mask
        idx_v[pl.ds(1 * L, L)] = (lx ^ hy0 ^ hz1) & mask
        idx_v[pl.ds(2 * L, L)] = (lx ^ hy1 ^ hz0) & mask
        idx_v[pl.ds(3 * L, L)] = (lx ^ hy1 ^ hz1) & mask
        idx_v[pl.ds(4 * L, L)] = (hx1 ^ hy0 ^ hz0) & mask
        idx_v[pl.ds(5 * L, L)] = (hx1 ^ hy0 ^ hz1) & mask
        idx_v[pl.ds(6 * L, L)] = (hx1 ^ hy1 ^ hz0) & mask
        idx_v[pl.ds(7 * L, L)] = (hx1 ^ hy1 ^ hz1) & mask
        one = jnp.int32(1)
        for c in range(8):
            pidx_v[pl.ds(c * L, L)] = idx_v[pl.ds(c * L, L)] >> one
        pltpu.async_copy(table_hbm.at[pidx_v], rows_v, sem)

    def wait_gather(pidx_v, rows_v, sem):
        pltpu.make_async_copy(table_hbm.at[pidx_v], rows_v, sem).wait()

    def interp(i, idx_v, rows_v):
        ob, l, _, _, _, fx, fy, fz = coords_frac(i)
        gx = jnp.float32(1.0) - fx
        gy = jnp.float32(1.0) - fy
        gz = jnp.float32(1.0) - fz
        w = [gx * gy * gz, gx * gy * fz, gx * fy * gz, gx * fy * fz,
             fx * gy * gz, fx * gy * fz, fx * fy * gz, fx * fy * fz]
        out_base = (iota + ob) * 256 + l * FEAT_DIM
        one = jnp.int32(1)
        colb = [(idx_v[pl.ds(c * L, L)] & one) * jnp.int32(FEAT_DIM)
                for c in range(8)]
        for f in range(FEAT_DIM):
            acc = w[0] * plsc.load_gather(rows_v, [iota, colb[0] + f])
            for c in range(1, 8):
                acc = acc + w[c] * plsc.load_gather(
                    rows_v, [iota + c * L, colb[c] + f])
            plsc.store_scatter(feat_v, [out_base + f], acc)

    def group_body(j, carry):
        for b in range(SLOTS):
            i = j * jnp.int32(SLOTS) + jnp.int32(b)
            sf = (b + AHEAD) % SLOTS
            wait_gather(pidx_vs[b], rows_vs[b], sems[b])
            compute_fire(i + jnp.int32(AHEAD), idx_vs[sf], pidx_vs[sf],
                         rows_vs[sf], sems[sf])
            interp(i, idx_vs[b], rows_vs[b])
        return carry

    n_groups = ITERS // SLOTS

    def chunk_body(k, carry):
        base = wbase + k * jnp.int32(CH)
        pltpu.sync_copy(coords_hbm.at[:, pl.ds(base, CH)], coords_v)
        for p in range(AHEAD):
            compute_fire(p, idx_vs[p], pidx_vs[p], rows_vs[p], sems[p])
        lax.fori_loop(jnp.int32(0), jnp.int32(n_groups - 1), group_body,
                      jnp.int32(0), unroll=False)
        for b in range(SLOTS):
            i = (n_groups - 1) * SLOTS + b
            sf = (b + AHEAD) % SLOTS
            wait_gather(pidx_vs[b], rows_vs[b], sems[b])
            if i + AHEAD < ITERS:
                compute_fire(i + AHEAD, idx_vs[sf], pidx_vs[sf], rows_vs[sf],
                             sems[sf])
            interp(i, idx_vs[b], rows_vs[b])
        pltpu.sync_copy(feat_v,
                        feat_hbm.at[pl.ds(base * jnp.int32(256), CH * 256)])
        return carry

    lax.fori_loop(jnp.int32(0), jnp.int32(N_CHUNKS), chunk_body,
                  jnp.int32(0), unroll=False)


def _sc_hashgrid(coords, table, nl_arr):
    mesh = plsc.VectorSubcoreMesh(core_axis_name="c", subcore_axis_name="s",
                                  num_cores=NC, num_subcores=NS)
    f = pl.kernel(
        _sc_hashgrid_body,
        out_type=jax.ShapeDtypeStruct((N_POINTS * 256,), jnp.float32),
        mesh=mesh,
        scratch_types=[
            pltpu.VMEM((3, CH), jnp.float32),
            pltpu.VMEM((GRID_LEVELS,), jnp.float32),
            [pltpu.VMEM((8 * L,), jnp.int32) for _ in range(SLOTS)],
            [pltpu.VMEM((8 * L,), jnp.int32) for _ in range(SLOTS)],
            [pltpu.VMEM((8 * L, 2 * FEAT_DIM), jnp.float32)
             for _ in range(SLOTS)],
            pltpu.VMEM((CH * 256,), jnp.float32),
            [pltpu.SemaphoreType.DMA for _ in range(SLOTS)],
        ],
        compiler_params=pltpu.CompilerParams(needs_layout_passes=False,
                                             use_tc_tiling_on_sc=False),
    )
    return f(coords, table, nl_arr)


def _posenc_body(x_ref, out_ref):
    v = x_ref[...]
    pi = jnp.float32(np.pi)
    s = jnp.sin(v * pi)
    c = jnp.cos(v * pi)
    out_ref[0] = v
    for i in range(6):
        out_ref[1 + 2 * i] = s
        out_ref[2 + 2 * i] = c
        if i < 5:
            s, c = jnp.float32(2.0) * s * c, jnp.float32(1.0) - jnp.float32(2.0) * s * s


def _posenc(x_flat):
    rows = x_flat.shape[0] // 128
    return pl.pallas_call(
        _posenc_body,
        out_shape=jax.ShapeDtypeStruct((13, rows, 128), jnp.float32),
    )(x_flat.reshape(rows, 128))


def kernel(x, t, mask, table_bank):
    msk = jnp.squeeze(mask)
    num_keep = msk.shape[0] - 1
    keep_idx = jnp.argsort(msk == 0)[:num_keep]
    x_sel = jnp.take(x, keep_idx, axis=-1)
    tidx = jnp.argmax(msk == 0)
    table = jnp.take(table_bank, tidx, axis=0).astype(jnp.float32)
    N, H, W = x_sel.shape[0], x_sel.shape[1], x_sel.shape[2]
    tt = jnp.broadcast_to(t[:, None, None, :], (N, H, W, 1)).astype(jnp.float32)
    x_t = jnp.concatenate([x_sel.astype(jnp.float32), tt], axis=-1)
    xt2 = x_t.reshape(N_POINTS, 3)
    coords = xt2.T.copy()
    nl_arr = jnp.asarray(_NL, dtype=jnp.float32)

    table2 = table.reshape(MAX_GRID // 2, 2 * FEAT_DIM)
    feat = _sc_hashgrid(coords, table2, nl_arr).reshape(N_POINTS, 256)
    enc = _posenc(xt2.reshape(-1))
    enc = enc.reshape(13, N_POINTS, 3).transpose(1, 0, 2).reshape(N_POINTS, 39)
    latent = jnp.concatenate([feat, enc], axis=-1)
    return latent.reshape(N, H, W, 256 + 39)
